# initial kernel scaffold (unmeasured)
import jax
import jax.numpy as jnp
from jax import lax
from jax.experimental import pallas as pl
from jax.experimental.pallas import tpu as pltpu

N_DEV = 4
B = 2
S_LOC = 512
D_MODEL = 768
HQ = 8
DH = 64
BLK = 64
NEG = -1e9


def kernel(x, Wq, K_ext, V_ext, Wo):
    def body(x_ref, wq_ref, k_ref, v_ref, wo_ref, out_ref,
             k_buf, v_buf, k_send, k_recv, v_send, v_recv):
        my_pos = lax.axis_index("i")
        left = lax.rem(my_pos + N_DEV - 1, N_DEV)
        right = lax.rem(my_pos + 1, N_DEV)

        barrier_sem = pltpu.get_barrier_semaphore()
        for nbr in (left, right):
            pl.semaphore_signal(
                barrier_sem, inc=1,
                device_id=(nbr,), device_id_type=pl.DeviceIdType.MESH,
            )
        pl.semaphore_wait(barrier_sem, 2)

        k_buf[0] = k_ref[...].astype(jnp.bfloat16)
        v_buf[0] = v_ref[...].astype(jnp.bfloat16)

        xb = x_ref[...].astype(jnp.bfloat16).reshape(B * S_LOC, D_MODEL)
        q = lax.dot_general(
            xb, wq_ref[...].astype(jnp.bfloat16),
            (((1,), (0,)), ((), ())),
            preferred_element_type=jnp.float32,
        )
        q = q.reshape(B, S_LOC, HQ, DH).astype(jnp.bfloat16)

        qb = (my_pos * S_LOC
              + lax.broadcasted_iota(jnp.int32, (S_LOC, 1), 0)) // BLK

        m = jnp.full((B, HQ, S_LOC, 1), NEG, jnp.float32)
        l = jnp.zeros((B, HQ, S_LOC, 1), jnp.float32)
        acc = jnp.zeros((B, HQ, S_LOC, DH), jnp.float32)

        for h in range(N_DEV):
            rdmas = []
            if h < N_DEV - 1:
                for buf, ss, rs in ((k_buf, k_send, k_recv),
                                    (v_buf, v_send, v_recv)):
                    rdma = pltpu.make_async_remote_copy(
                        src_ref=buf.at[h],
                        dst_ref=buf.at[h + 1],
                        send_sem=ss.at[h],
                        recv_sem=rs.at[h],
                        device_id=(right,),
                        device_id_type=pl.DeviceIdType.MESH,
                    )
                    rdma.start()
                    rdmas.append(rdma)

            origin = lax.rem(my_pos + N_DEV - h, N_DEV)
            kc = k_buf[h]
            vc = v_buf[h]

            scores = lax.dot_general(
                q, kc, (((3,), (3,)), ((0, 2), (0, 2))),
                preferred_element_type=jnp.float32,
            ) * 0.125

            kb = (origin * S_LOC
                  + lax.broadcasted_iota(jnp.int32, (1, S_LOC), 1)) // BLK
            mask = ((qb == kb) | (kb == 0)
                    | (lax.rem(qb + kb, 3) == 0))[None, None]
            scores = jnp.where(mask, scores, NEG)

            m_new = jnp.maximum(m, jnp.max(scores, axis=-1, keepdims=True))
            alpha = jnp.exp(m - m_new)
            p = jnp.where(mask, jnp.exp(scores - m_new), 0.0)
            l = l * alpha + jnp.sum(p, axis=-1, keepdims=True)
            pv = lax.dot_general(
                p.astype(jnp.bfloat16), vc,
                (((3,), (1,)), ((0, 1), (0, 2))),
                preferred_element_type=jnp.float32,
            )
            acc = acc * alpha + pv

            for rdma in rdmas:
                rdma.wait()

        ctx = (acc / l).astype(jnp.bfloat16)
        ctx = jnp.transpose(ctx, (0, 2, 1, 3)).reshape(B * S_LOC, HQ * DH)
        out = lax.dot_general(
            ctx, wo_ref[...].astype(jnp.bfloat16),
            (((1,), (0,)), ((), ())),
            preferred_element_type=jnp.float32,
        )
        out_ref[...] = out.reshape(B, S_LOC, D_MODEL)

    return pl.pallas_call(
        body,
        out_shape=jax.ShapeDtypeStruct((B, S_LOC, D_MODEL), jnp.float32),
        in_specs=[pl.BlockSpec(memory_space=pltpu.VMEM)] * 5,
        out_specs=pl.BlockSpec(memory_space=pltpu.VMEM),
        scratch_shapes=[
            pltpu.VMEM((N_DEV, B, S_LOC, HQ, DH), jnp.bfloat16),
            pltpu.VMEM((N_DEV, B, S_LOC, HQ, DH), jnp.bfloat16),
            pltpu.SemaphoreType.DMA((N_DEV - 1,)),
            pltpu.SemaphoreType.DMA((N_DEV - 1,)),
            pltpu.SemaphoreType.DMA((N_DEV - 1,)),
            pltpu.SemaphoreType.DMA((N_DEV - 1,)),
        ],
        compiler_params=pltpu.CompilerParams(collective_id=0),
    )(x, Wq, K_ext, V_ext, Wo)


# baseline (device time: 123496 ns/iter reference)
import jax
import jax.numpy as jnp
from jax import lax
from jax.experimental import pallas as pl
from jax.experimental.pallas import tpu as pltpu

N_DEV = 4
B = 2
S_LOC = 512
D_MODEL = 768
HQ = 8
DH = 64
BH = B * HQ
HG = 8
NG = BH // HG
BLK = 64
NEG = -1e9


def kernel(x, Wq, K_ext, V_ext, Wo):
    def body(x_ref, wq_ref, k_ref, v_ref, wo_ref, out_ref,
             k_buf, v_buf, k_send, k_recv, v_send, v_recv):
        my_pos = lax.axis_index("i")
        left = lax.rem(my_pos + N_DEV - 1, N_DEV)
        right = lax.rem(my_pos + 1, N_DEV)

        barrier_sem = pltpu.get_barrier_semaphore()
        for nbr in (left, right):
            pl.semaphore_signal(
                barrier_sem, inc=1,
                device_id=(nbr,), device_id_type=pl.DeviceIdType.MESH,
            )
        pl.semaphore_wait(barrier_sem, 2)

        def to_t(a):
            return jnp.transpose(a.astype(jnp.bfloat16), (0, 2, 3, 1)
                                 ).reshape(BH, DH, S_LOC)

        k_buf[0] = to_t(k_ref[...])
        v_buf[0] = to_t(v_ref[...])

        xb = x_ref[...].astype(jnp.bfloat16).reshape(B * S_LOC, D_MODEL)
        q = lax.dot_general(
            xb, wq_ref[...].astype(jnp.bfloat16),
            (((1,), (0,)), ((), ())),
            preferred_element_type=jnp.float32,
        )
        q = jnp.transpose(
            q.reshape(B, S_LOC, HQ, DH).astype(jnp.bfloat16), (0, 2, 1, 3)
        ).reshape(BH, S_LOC, DH)
        qs = [q[g * HG:(g + 1) * HG] for g in range(NG)]

        qb = (my_pos * S_LOC
              + lax.broadcasted_iota(jnp.int32, (S_LOC, 1), 0)) // BLK

        ms = [jnp.full((HG, S_LOC, 1), NEG, jnp.float32) for _ in range(NG)]
        ls = [jnp.zeros((HG, S_LOC, 1), jnp.float32) for _ in range(NG)]
        accs = [jnp.zeros((HG, S_LOC, DH), jnp.float32) for _ in range(NG)]

        for h in range(N_DEV):
            rdmas = []
            if h < N_DEV - 1:
                for buf, ss, rs in ((k_buf, k_send, k_recv),
                                    (v_buf, v_send, v_recv)):
                    rdma = pltpu.make_async_remote_copy(
                        src_ref=buf.at[h],
                        dst_ref=buf.at[h + 1],
                        send_sem=ss.at[h],
                        recv_sem=rs.at[h],
                        device_id=(right,),
                        device_id_type=pl.DeviceIdType.MESH,
                    )
                    rdma.start()
                    rdmas.append(rdma)

            origin = lax.rem(my_pos + N_DEV - h, N_DEV)
            kb = (origin * S_LOC
                  + lax.broadcasted_iota(jnp.int32, (1, S_LOC), 1)) // BLK
            mask = ((qb == kb) | (kb == 0)
                    | (lax.rem(qb + kb, 3) == 0))[None]

            for g in range(NG):
                kc = k_buf[h, g * HG:(g + 1) * HG]
                vc = v_buf[h, g * HG:(g + 1) * HG]

                scores = lax.dot_general(
                    qs[g], kc, (((2,), (1,)), ((0,), (0,))),
                    preferred_element_type=jnp.float32,
                ) * 0.125
                scores = jnp.where(mask, scores, NEG)

                m_new = jnp.maximum(
                    ms[g], jnp.max(scores, axis=-1, keepdims=True))
                alpha = jnp.exp(ms[g] - m_new)
                p = jnp.where(mask, jnp.exp(scores - m_new), 0.0)
                ls[g] = ls[g] * alpha + jnp.sum(p, axis=-1, keepdims=True)
                pv = lax.dot_general(
                    p.astype(jnp.bfloat16), vc,
                    (((2,), (2,)), ((0,), (0,))),
                    preferred_element_type=jnp.float32,
                )
                accs[g] = accs[g] * alpha + pv
                ms[g] = m_new

            for rdma in rdmas:
                rdma.wait()

        ctx = jnp.concatenate(
            [(accs[g] / ls[g]).astype(jnp.bfloat16) for g in range(NG)], axis=0
        ).reshape(B, HQ, S_LOC, DH)
        ctx = jnp.transpose(ctx, (0, 2, 1, 3)).reshape(B * S_LOC, HQ * DH)
        out = lax.dot_general(
            ctx, wo_ref[...].astype(jnp.bfloat16),
            (((1,), (0,)), ((), ())),
            preferred_element_type=jnp.float32,
        )
        out_ref[...] = out.reshape(B, S_LOC, D_MODEL)

    return pl.pallas_call(
        body,
        out_shape=jax.ShapeDtypeStruct((B, S_LOC, D_MODEL), jnp.float32),
        in_specs=[pl.BlockSpec(memory_space=pltpu.VMEM)] * 5,
        out_specs=pl.BlockSpec(memory_space=pltpu.VMEM),
        scratch_shapes=[
            pltpu.VMEM((N_DEV, BH, DH, S_LOC), jnp.bfloat16),
            pltpu.VMEM((N_DEV, BH, DH, S_LOC), jnp.bfloat16),
            pltpu.SemaphoreType.DMA((N_DEV - 1,)),
            pltpu.SemaphoreType.DMA((N_DEV - 1,)),
            pltpu.SemaphoreType.DMA((N_DEV - 1,)),
            pltpu.SemaphoreType.DMA((N_DEV - 1,)),
        ],
        compiler_params=pltpu.CompilerParams(
            collective_id=0, vmem_limit_bytes=110 * 1024 * 1024,
        ),
    )(x, Wq, K_ext, V_ext, Wo)


# device time: 119696 ns/iter; 1.0317x vs baseline; 1.0317x over previous
import jax
import jax.numpy as jnp
from jax import lax
from jax.experimental import pallas as pl
from jax.experimental.pallas import tpu as pltpu

N_DEV = 4
B = 2
S_LOC = 512
D_MODEL = 768
HQ = 8
DH = 64
BH = B * HQ
HG = 8
NG = BH // HG
BLK = 64
NEG = -1e9


def kernel(x, Wq, K_ext, V_ext, Wo):
    def body(x_ref, wq_ref, k_ref, v_ref, wo_ref, out_ref,
             k_buf, v_buf, k_send, k_recv, v_send, v_recv):
        my_pos = lax.axis_index("i")
        left = lax.rem(my_pos + N_DEV - 1, N_DEV)
        right = lax.rem(my_pos + 1, N_DEV)

        barrier_sem = pltpu.get_barrier_semaphore()
        for nbr in (left, right):
            pl.semaphore_signal(
                barrier_sem, inc=1,
                device_id=(nbr,), device_id_type=pl.DeviceIdType.MESH,
            )
        pl.semaphore_wait(barrier_sem, 2)

        def to_t(a):
            return jnp.transpose(a.astype(jnp.bfloat16), (0, 2, 3, 1)
                                 ).reshape(BH, DH, S_LOC)

        k_buf[0] = to_t(k_ref[...])
        v_buf[0] = to_t(v_ref[...])

        xb = x_ref[...].astype(jnp.bfloat16).reshape(B * S_LOC, D_MODEL)
        q = lax.dot_general(
            xb, wq_ref[...].astype(jnp.bfloat16),
            (((1,), (0,)), ((), ())),
            preferred_element_type=jnp.float32,
        )
        q = jnp.transpose(
            q.reshape(B, S_LOC, HQ, DH).astype(jnp.bfloat16), (0, 2, 1, 3)
        ).reshape(BH, S_LOC, DH)
        qs = [q[g * HG:(g + 1) * HG] for g in range(NG)]

        qb = (my_pos * S_LOC
              + lax.broadcasted_iota(jnp.int32, (S_LOC, 1), 0)) // BLK

        ls = [jnp.zeros((HG, S_LOC, 1), jnp.float32) for _ in range(NG)]
        accs = [jnp.zeros((HG, S_LOC, DH), jnp.float32) for _ in range(NG)]

        for h in range(N_DEV):
            rdmas = []
            if h < N_DEV - 1:
                for buf, ss, rs in ((k_buf, k_send, k_recv),
                                    (v_buf, v_send, v_recv)):
                    rdma = pltpu.make_async_remote_copy(
                        src_ref=buf.at[h],
                        dst_ref=buf.at[h + 1],
                        send_sem=ss.at[h],
                        recv_sem=rs.at[h],
                        device_id=(right,),
                        device_id_type=pl.DeviceIdType.MESH,
                    )
                    rdma.start()
                    rdmas.append(rdma)

            origin = lax.rem(my_pos + N_DEV - h, N_DEV)
            kb = (origin * S_LOC
                  + lax.broadcasted_iota(jnp.int32, (1, S_LOC), 1)) // BLK
            mask = ((qb == kb) | (kb == 0)
                    | (lax.rem(qb + kb, 3) == 0))[None]
            bias = jnp.where(mask, 0.0, NEG)

            for g in range(NG):
                kc = k_buf[h, g * HG:(g + 1) * HG]
                vc = v_buf[h, g * HG:(g + 1) * HG]

                scores = lax.dot_general(
                    qs[g], kc, (((2,), (1,)), ((0,), (0,))),
                    preferred_element_type=jnp.float32,
                )
                p = jnp.exp(scores * 0.125 + bias)
                ls[g] = ls[g] + jnp.sum(p, axis=-1, keepdims=True)
                pv = lax.dot_general(
                    p.astype(jnp.bfloat16), vc,
                    (((2,), (2,)), ((0,), (0,))),
                    preferred_element_type=jnp.float32,
                )
                accs[g] = accs[g] + pv

            for rdma in rdmas:
                rdma.wait()

        ctx = jnp.concatenate(
            [(accs[g] / ls[g]).astype(jnp.bfloat16) for g in range(NG)], axis=0
        ).reshape(B, HQ, S_LOC, DH)
        ctx = jnp.transpose(ctx, (0, 2, 1, 3)).reshape(B * S_LOC, HQ * DH)
        out = lax.dot_general(
            ctx, wo_ref[...].astype(jnp.bfloat16),
            (((1,), (0,)), ((), ())),
            preferred_element_type=jnp.float32,
        )
        out_ref[...] = out.reshape(B, S_LOC, D_MODEL)

    return pl.pallas_call(
        body,
        out_shape=jax.ShapeDtypeStruct((B, S_LOC, D_MODEL), jnp.float32),
        in_specs=[pl.BlockSpec(memory_space=pltpu.VMEM)] * 5,
        out_specs=pl.BlockSpec(memory_space=pltpu.VMEM),
        scratch_shapes=[
            pltpu.VMEM((N_DEV, BH, DH, S_LOC), jnp.bfloat16),
            pltpu.VMEM((N_DEV, BH, DH, S_LOC), jnp.bfloat16),
            pltpu.SemaphoreType.DMA((N_DEV - 1,)),
            pltpu.SemaphoreType.DMA((N_DEV - 1,)),
            pltpu.SemaphoreType.DMA((N_DEV - 1,)),
            pltpu.SemaphoreType.DMA((N_DEV - 1,)),
        ],
        compiler_params=pltpu.CompilerParams(
            collective_id=0, vmem_limit_bytes=110 * 1024 * 1024,
        ),
    )(x, Wq, K_ext, V_ext, Wo)
